# baseline (speedup 1.0000x reference)
import functools

import jax
import jax.numpy as jnp
from jax import lax
from jax.experimental import pallas as pl
from jax.experimental.pallas import tpu as pltpu
from jax.experimental.pallas import tpu_sc as plsc

NUM_EXPERTS = 64
TOP_K = 8
DIM = 1024
HIDDEN_DIM = 512
ROUTE_SCALE = 1.0

NC = 2
NS = 16
NW = NC * NS

BR = 128


def _router_body(x_ref, gwt_ref, bias_ref, sel_ref, w_ref, rank_ref,
                 counts_ref, carry_ref):
    tb = x_ref.shape[0]

    @pl.when(pl.program_id(0) == 0)
    def _():
        carry_ref[...] = jnp.zeros_like(carry_ref)

    xb = x_ref[...]
    scores = jax.nn.sigmoid(
        jnp.dot(xb, gwt_ref[...], preferred_element_type=jnp.float32))
    biased = scores + bias_ref[0:1, :]
    iota_e = lax.broadcasted_iota(jnp.int32, (tb, NUM_EXPERTS), 1)

    cur = biased
    msum = jnp.zeros((tb, NUM_EXPERTS), jnp.float32)
    sel_ks, sc_ks, oh_ks = [], [], []
    for _ in range(TOP_K):
        m = jnp.max(cur, axis=1, keepdims=True)
        idx = jnp.min(jnp.where(cur == m, iota_e, NUM_EXPERTS), axis=1,
                      keepdims=True)
        onehot = iota_e == idx
        sel_ks.append(idx[:, 0])
        sc_ks.append(jnp.sum(jnp.where(onehot, scores, 0.0), axis=1))
        oh_ks.append(onehot)
        msum = msum + onehot.astype(jnp.float32)
        cur = jnp.where(onehot, -jnp.inf, cur)

    sc = jnp.stack(sc_ks, axis=0)
    denom = jnp.maximum(jnp.sum(sc, axis=0, keepdims=True), 1e-20)
    w_ref[...] = sc / denom * ROUTE_SCALE
    sel_ref[...] = jnp.stack(sel_ks, axis=0).astype(jnp.int32)

    carry0 = carry_ref[0:1, :].astype(jnp.float32)
    tri = (lax.broadcasted_iota(jnp.int32, (tb, tb), 0)
           >= lax.broadcasted_iota(jnp.int32, (tb, tb), 1)).astype(jnp.float32)
    cum = jnp.dot(tri, msum, preferred_element_type=jnp.float32)
    c_excl = carry0 + cum - msum
    ranks = [jnp.sum(jnp.where(oh_ks[k], c_excl, 0), axis=1)
             for k in range(TOP_K)]
    rank_ref[...] = jnp.stack(ranks, axis=0).astype(jnp.int32)
    new_carry = jnp.broadcast_to(carry0 + cum[tb - 1:tb, :],
                                 (8, NUM_EXPERTS)).astype(jnp.int32)
    carry_ref[...] = new_carry
    counts_ref[...] = new_carry


def _router(xf, gwt, bias8):
    t = xf.shape[0]
    tb = 512
    grid = (t // tb,)
    return pl.pallas_call(
        _router_body,
        grid=grid,
        in_specs=[
            pl.BlockSpec((tb, DIM), lambda i: (i, 0)),
            pl.BlockSpec((DIM, NUM_EXPERTS), lambda i: (0, 0)),
            pl.BlockSpec((8, NUM_EXPERTS), lambda i: (0, 0)),
        ],
        out_specs=[
            pl.BlockSpec((TOP_K, tb), lambda i: (0, i)),
            pl.BlockSpec((TOP_K, tb), lambda i: (0, i)),
            pl.BlockSpec((TOP_K, tb), lambda i: (0, i)),
            pl.BlockSpec((8, NUM_EXPERTS), lambda i: (0, 0)),
        ],
        out_shape=[
            jax.ShapeDtypeStruct((TOP_K, t), jnp.int32),
            jax.ShapeDtypeStruct((TOP_K, t), jnp.float32),
            jax.ShapeDtypeStruct((TOP_K, t), jnp.int32),
            jax.ShapeDtypeStruct((8, NUM_EXPERTS), jnp.int32),
        ],
        scratch_shapes=[pltpu.VMEM((8, NUM_EXPERTS), jnp.int32)],
    )(xf, gwt, bias8)


def _destcalc_body(e_ref, r_ref, w_ref, off_ref, dest_ref, w16_ref):
    e = e_ref[...]
    acc = r_ref[...]
    for j in range(NUM_EXPERTS):
        acc = acc + jnp.where(e == j, off_ref[j], 0)
    dest_ref[...] = acc
    w16_ref[...] = jnp.broadcast_to(w_ref[...], w16_ref.shape)


def _destcalc(e_t, r_t, w2, offset_pad):
    k, t = e_t.shape
    tb = t // 8
    tk = k * t
    tkb = tk // 8
    return pl.pallas_call(
        _destcalc_body,
        grid=(8,),
        in_specs=[
            pl.BlockSpec((k, tb), lambda i: (0, i)),
            pl.BlockSpec((k, tb), lambda i: (0, i)),
            pl.BlockSpec((tkb, 1), lambda i: (i, 0)),
            pl.BlockSpec(memory_space=pltpu.SMEM),
        ],
        out_specs=[
            pl.BlockSpec((k, tb), lambda i: (0, i)),
            pl.BlockSpec((tkb, 128), lambda i: (i, 0)),
        ],
        out_shape=[
            jax.ShapeDtypeStruct((k, t), jnp.int32),
            jax.ShapeDtypeStruct((tk, 128), jnp.float32),
        ],
    )(e_t, r_t, w2, offset_pad)


def _dispatch(xf, dest_t, w16, cap):
    t = xf.shape[0]
    tok_per = t // NW
    tch = 64
    nch = tok_per // tch
    mesh = plsc.VectorSubcoreMesh(core_axis_name="c", subcore_axis_name="s",
                                  num_cores=NC, num_subcores=NS)

    @functools.partial(
        pl.kernel,
        out_type=[
            jax.ShapeDtypeStruct((cap, DIM), jnp.float32),
            jax.ShapeDtypeStruct((cap, 128), jnp.float32),
        ],
        mesh=mesh,
        scratch_types=[
            pltpu.VMEM((TOP_K, tch), jnp.int32),
            pltpu.VMEM((tch, 128), jnp.float32),
            pltpu.VMEM((tch, 128), jnp.float32),
            pltpu.VMEM((tch, DIM), jnp.float32),
            pltpu.SemaphoreType.DMA,
            pltpu.SemaphoreType.DMA,
        ],
    )
    def dispatch(xf_hbm, dt_hbm, w16_hbm, perm_hbm, s16_hbm,
                 idx_v, s16a_v, s16b_v, rows_v, sem, sem2):
        wid = lax.axis_index("s") * NC + lax.axis_index("c")
        tbase = wid * tok_per

        def chunk_body(ci, carry):
            t0 = tbase + ci * tch
            pltpu.sync_copy(xf_hbm.at[pl.ds(t0, tch)], rows_v)
            for k in range(TOP_K):
                pltpu.sync_copy(dt_hbm.at[k, pl.ds(t0, tch)], idx_v.at[k])
            cps = []
            for k in range(TOP_K):
                cps.append(
                    pltpu.async_copy(rows_v, perm_hbm.at[idx_v.at[k]], sem))
            bufs = (s16a_v, s16b_v)
            cps2 = []
            for k in range(TOP_K):
                sb = bufs[k % 2]
                if k >= 2:
                    cps2[k - 2].wait()
                pltpu.sync_copy(w16_hbm.at[pl.ds(k * t + t0, tch)], sb)
                cps2.append(
                    pltpu.async_copy(sb, s16_hbm.at[idx_v.at[k]], sem2))
            cps2[TOP_K - 2].wait()
            cps2[TOP_K - 1].wait()
            for cp in cps:
                cp.wait()
            return carry

        lax.fori_loop(0, nch, chunk_body, 0)

    return dispatch(xf, dest_t, w16)


def _expert_body(blk_ref, p_ref, s_ref, w1_ref, w3_ref, w2_ref, o_ref):
    p = (p_ref[...] * s_ref[:, 0:1]).astype(jnp.bfloat16)
    w1b = w1_ref[0].astype(jnp.bfloat16)
    w3b = w3_ref[0].astype(jnp.bfloat16)
    a = jnp.dot(p, w1b, preferred_element_type=jnp.float32)
    b = jnp.dot(p, w3b, preferred_element_type=jnp.float32)
    h = (a * jax.nn.sigmoid(a) * b).astype(jnp.bfloat16)
    o_ref[...] = jnp.dot(h, w2_ref[0].astype(jnp.bfloat16),
                         preferred_element_type=jnp.float32)


def _experts(blk_expert, perm, s16, w1, w2, w3, nblk):
    grid_spec = pltpu.PrefetchScalarGridSpec(
        num_scalar_prefetch=1,
        grid=(nblk,),
        in_specs=[
            pl.BlockSpec((BR, DIM), lambda i, blk: (i, 0)),
            pl.BlockSpec((BR, 128), lambda i, blk: (i, 0)),
            pl.BlockSpec((1, DIM, HIDDEN_DIM), lambda i, blk: (blk[i], 0, 0)),
            pl.BlockSpec((1, DIM, HIDDEN_DIM), lambda i, blk: (blk[i], 0, 0)),
            pl.BlockSpec((1, HIDDEN_DIM, DIM), lambda i, blk: (blk[i], 0, 0)),
        ],
        out_specs=pl.BlockSpec((BR, DIM), lambda i, blk: (i, 0)),
    )
    return pl.pallas_call(
        _expert_body,
        grid_spec=grid_spec,
        out_shape=jax.ShapeDtypeStruct((nblk * BR, DIM), jnp.float32),
    )(blk_expert, perm, s16, w1, w3, w2)


def _combine(eo, dest_t, t):
    tok_per = t // NW
    tch = 8
    nch = tok_per // tch
    mesh = plsc.VectorSubcoreMesh(core_axis_name="c", subcore_axis_name="s",
                                  num_cores=NC, num_subcores=NS)

    @functools.partial(
        pl.kernel,
        out_type=jax.ShapeDtypeStruct((t, DIM), jnp.float32),
        mesh=mesh,
        scratch_types=[
            pltpu.VMEM((TOP_K, tok_per), jnp.int32),
            pltpu.VMEM((TOP_K * tch, DIM), jnp.float32),
            pltpu.VMEM((tch, DIM), jnp.float32),
            pltpu.SemaphoreType.DMA,
        ],
    )
    def combine(eo_hbm, dt_hbm, out_hbm, idx_v, rows_v, out_v, sem):
        wid = lax.axis_index("s") * NC + lax.axis_index("c")
        tbase = wid * tok_per
        for k in range(TOP_K):
            pltpu.sync_copy(dt_hbm.at[k, pl.ds(tbase, tok_per)], idx_v.at[k])

        def chunk_body(ci, carry):
            c0 = ci * tch
            cps = []
            for k in range(TOP_K):
                cps.append(pltpu.async_copy(
                    eo_hbm.at[idx_v.at[k, pl.ds(c0, tch)]],
                    rows_v.at[pl.ds(k * tch, tch)], sem))
            for cp in cps:
                cp.wait()

            def cbody(c, c2):
                sl = pl.ds(c * 16, 16)
                for tt in range(tch):
                    acc = rows_v[tt, sl]
                    for j in range(1, TOP_K):
                        acc = acc + rows_v[j * tch + tt, sl]
                    out_v[tt, sl] = acc
                return c2

            lax.fori_loop(0, DIM // 16, cbody, 0)
            pltpu.sync_copy(out_v, out_hbm.at[pl.ds(tbase + c0, tch)])
            return carry

        lax.fori_loop(0, nch, chunk_body, 0)

    return combine(eo, dest_t)


def kernel(x, gate_w, w1, w2, w3, expert_bias):
    bs, slen, dim = x.shape
    xf = x.reshape(-1, dim).astype(jnp.float32)
    t = xf.shape[0]
    tk = t * TOP_K
    nblk = tk // BR + NUM_EXPERTS
    cap = nblk * BR

    gwt = gate_w.T
    bias8 = jnp.broadcast_to(expert_bias[None, :], (8, NUM_EXPERTS))

    sel_t, w_t, rank_t, counts8 = _router(xf, gwt, bias8)

    counts = counts8[0]
    nblk_e = (counts + BR - 1) // BR
    offset_pad = ((jnp.cumsum(nblk_e) - nblk_e) * BR).astype(jnp.int32)
    blk_expert = jnp.repeat(
        jnp.arange(NUM_EXPERTS, dtype=jnp.int32), nblk_e,
        total_repeat_length=nblk)

    dest_t, w16 = _destcalc(sel_t, rank_t, w_t.reshape(-1, 1), offset_pad)

    perm, s16 = _dispatch(xf, dest_t, w16, cap)
    eo = _experts(blk_expert, perm, s16, w1, w2, w3, nblk)
    out = _combine(eo, dest_t, t)
    return out.reshape(bs, slen, dim)


# speedup vs baseline: 1.0789x; 1.0789x over previous
import functools

import jax
import jax.numpy as jnp
from jax import lax
from jax.experimental import pallas as pl
from jax.experimental.pallas import tpu as pltpu
from jax.experimental.pallas import tpu_sc as plsc

NUM_EXPERTS = 64
TOP_K = 8
DIM = 1024
HIDDEN_DIM = 512
ROUTE_SCALE = 1.0

NC = 2
NS = 16
NW = NC * NS

BR = 128


def _router_body(x_ref, gwt_ref, bias_ref, sel_ref, w_ref, rank_ref,
                 counts_ref, carry_ref):
    tb = x_ref.shape[0]

    @pl.when(pl.program_id(0) == 0)
    def _():
        carry_ref[...] = jnp.zeros_like(carry_ref)

    xb = x_ref[...]
    scores = jax.nn.sigmoid(
        jnp.dot(xb, gwt_ref[...], preferred_element_type=jnp.float32))
    biased = scores + bias_ref[0:1, :]
    iota_e = lax.broadcasted_iota(jnp.int32, (tb, NUM_EXPERTS), 1)

    cur = biased
    msum = jnp.zeros((tb, NUM_EXPERTS), jnp.float32)
    sel_ks, sc_ks, oh_ks = [], [], []
    for _ in range(TOP_K):
        m = jnp.max(cur, axis=1, keepdims=True)
        idx = jnp.min(jnp.where(cur == m, iota_e, NUM_EXPERTS), axis=1,
                      keepdims=True)
        onehot = iota_e == idx
        sel_ks.append(idx[:, 0])
        sc_ks.append(jnp.sum(jnp.where(onehot, scores, 0.0), axis=1))
        oh_ks.append(onehot)
        msum = msum + onehot.astype(jnp.float32)
        cur = jnp.where(onehot, -jnp.inf, cur)

    sc = jnp.stack(sc_ks, axis=0)
    denom = jnp.maximum(jnp.sum(sc, axis=0, keepdims=True), 1e-20)
    w_ref[...] = sc / denom * ROUTE_SCALE
    sel_ref[...] = jnp.stack(sel_ks, axis=0).astype(jnp.int32)

    carry0 = carry_ref[0:1, :].astype(jnp.float32)
    tri = (lax.broadcasted_iota(jnp.int32, (tb, tb), 0)
           >= lax.broadcasted_iota(jnp.int32, (tb, tb), 1)).astype(jnp.float32)
    cum = jnp.dot(tri, msum, preferred_element_type=jnp.float32)
    c_excl = carry0 + cum - msum
    ranks = [jnp.sum(jnp.where(oh_ks[k], c_excl, 0), axis=1)
             for k in range(TOP_K)]
    rank_ref[...] = jnp.stack(ranks, axis=0).astype(jnp.int32)
    new_carry = jnp.broadcast_to(carry0 + cum[tb - 1:tb, :],
                                 (8, NUM_EXPERTS)).astype(jnp.int32)
    carry_ref[...] = new_carry
    counts_ref[...] = new_carry


def _router(xf, gwt, bias8):
    t = xf.shape[0]
    tb = 512
    grid = (t // tb,)
    return pl.pallas_call(
        _router_body,
        grid=grid,
        in_specs=[
            pl.BlockSpec((tb, DIM), lambda i: (i, 0)),
            pl.BlockSpec((DIM, NUM_EXPERTS), lambda i: (0, 0)),
            pl.BlockSpec((8, NUM_EXPERTS), lambda i: (0, 0)),
        ],
        out_specs=[
            pl.BlockSpec((TOP_K, tb), lambda i: (0, i)),
            pl.BlockSpec((TOP_K, tb), lambda i: (0, i)),
            pl.BlockSpec((TOP_K, tb), lambda i: (0, i)),
            pl.BlockSpec((8, NUM_EXPERTS), lambda i: (0, 0)),
        ],
        out_shape=[
            jax.ShapeDtypeStruct((TOP_K, t), jnp.int32),
            jax.ShapeDtypeStruct((TOP_K, t), jnp.float32),
            jax.ShapeDtypeStruct((TOP_K, t), jnp.int32),
            jax.ShapeDtypeStruct((8, NUM_EXPERTS), jnp.int32),
        ],
        scratch_shapes=[pltpu.VMEM((8, NUM_EXPERTS), jnp.int32)],
    )(xf, gwt, bias8)


def _destcalc_body(e_ref, r_ref, w_ref, off_ref, dest_ref, w16_ref):
    e = e_ref[...]
    acc = r_ref[...]
    for j in range(NUM_EXPERTS):
        acc = acc + jnp.where(e == j, off_ref[j], 0)
    dest_ref[...] = acc
    w16_ref[...] = jnp.broadcast_to(w_ref[...], w16_ref.shape)


def _destcalc(e_t, r_t, w2, offset_pad):
    k, t = e_t.shape
    tb = t // 8
    tk = k * t
    tkb = tk // 8
    return pl.pallas_call(
        _destcalc_body,
        grid=(8,),
        in_specs=[
            pl.BlockSpec((k, tb), lambda i: (0, i)),
            pl.BlockSpec((k, tb), lambda i: (0, i)),
            pl.BlockSpec((tkb, 1), lambda i: (i, 0)),
            pl.BlockSpec(memory_space=pltpu.SMEM),
        ],
        out_specs=[
            pl.BlockSpec((k, tb), lambda i: (0, i)),
            pl.BlockSpec((tkb, 128), lambda i: (i, 0)),
        ],
        out_shape=[
            jax.ShapeDtypeStruct((k, t), jnp.int32),
            jax.ShapeDtypeStruct((tk, 128), jnp.float32),
        ],
    )(e_t, r_t, w2, offset_pad)


def _dispatch(xf, dest_t, w16, cap):
    t = xf.shape[0]
    tok_per = t // NW
    tch = 64
    nch = tok_per // tch
    mesh = plsc.VectorSubcoreMesh(core_axis_name="c", subcore_axis_name="s",
                                  num_cores=NC, num_subcores=NS)

    @functools.partial(
        pl.kernel,
        out_type=[
            jax.ShapeDtypeStruct((cap, DIM), jnp.float32),
            jax.ShapeDtypeStruct((cap, 128), jnp.float32),
        ],
        mesh=mesh,
        scratch_types=[
            pltpu.VMEM((TOP_K, tch), jnp.int32),
            pltpu.VMEM((tch, 128), jnp.float32),
            pltpu.VMEM((tch, 128), jnp.float32),
            pltpu.VMEM((tch, DIM), jnp.float32),
            pltpu.SemaphoreType.DMA,
            pltpu.SemaphoreType.DMA,
        ],
    )
    def dispatch(xf_hbm, dt_hbm, w16_hbm, perm_hbm, s16_hbm,
                 idx_v, s16a_v, s16b_v, rows_v, sem, sem2):
        wid = lax.axis_index("s") * NC + lax.axis_index("c")
        tbase = wid * tok_per

        def chunk_body(ci, carry):
            t0 = tbase + ci * tch
            pltpu.sync_copy(xf_hbm.at[pl.ds(t0, tch)], rows_v)
            for k in range(TOP_K):
                pltpu.sync_copy(dt_hbm.at[k, pl.ds(t0, tch)], idx_v.at[k])
            cps = []
            for k in range(TOP_K):
                cps.append(
                    pltpu.async_copy(rows_v, perm_hbm.at[idx_v.at[k]], sem))
            bufs = (s16a_v, s16b_v)
            cps2 = []
            for k in range(TOP_K):
                sb = bufs[k % 2]
                if k >= 2:
                    cps2[k - 2].wait()
                pltpu.sync_copy(w16_hbm.at[pl.ds(k * t + t0, tch)], sb)
                cps2.append(
                    pltpu.async_copy(sb, s16_hbm.at[idx_v.at[k]], sem2))
            cps2[TOP_K - 2].wait()
            cps2[TOP_K - 1].wait()
            for cp in cps:
                cp.wait()
            return carry

        lax.fori_loop(0, nch, chunk_body, 0)

    return dispatch(xf, dest_t, w16)


def _expert_body(meta_ref, p_ref, s_ref, w1_hbm, w3_hbm, w2_hbm, o_ref,
                 w1s, w3s, w2s, sems):
    i = pl.program_id(0)
    first = meta_ref[0, i]
    slot = meta_ref[1, i]
    next_e = meta_ref[2, i]
    next_slot = meta_ref[3, i]
    cur_e = meta_ref[4, i]
    do_start = meta_ref[5, i]

    def _start(e, s):
        pltpu.make_async_copy(w1_hbm.at[e], w1s.at[s], sems.at[s]).start()
        pltpu.make_async_copy(w3_hbm.at[e], w3s.at[s], sems.at[s]).start()
        pltpu.make_async_copy(w2_hbm.at[e], w2s.at[s], sems.at[s]).start()

    def _wait(e, s):
        pltpu.make_async_copy(w1_hbm.at[e], w1s.at[s], sems.at[s]).wait()
        pltpu.make_async_copy(w3_hbm.at[e], w3s.at[s], sems.at[s]).wait()
        pltpu.make_async_copy(w2_hbm.at[e], w2s.at[s], sems.at[s]).wait()

    @pl.when(i == 0)
    def _():
        _start(cur_e, slot)

    @pl.when(first == 1)
    def _():
        _wait(cur_e, slot)

    @pl.when(do_start == 1)
    def _():
        _start(next_e, next_slot)

    p = p_ref[...] * s_ref[:, 0:1]
    a = jnp.dot(p, w1s[slot], preferred_element_type=jnp.float32)
    b = jnp.dot(p, w3s[slot], preferred_element_type=jnp.float32)
    h = a * jax.nn.sigmoid(a) * b
    o_ref[...] = jnp.dot(h, w2s[slot], preferred_element_type=jnp.float32)


def _experts(blk_expert, perm, s16, w1, w2, w3, nblk):
    diff = blk_expert[1:] != blk_expert[:-1]
    first_flag = jnp.concatenate(
        [jnp.ones((1,), jnp.int32), diff.astype(jnp.int32)])
    run_id = jnp.cumsum(first_flag) - 1
    total_runs = run_id[-1] + 1
    slot = run_id % 3
    next_slot = (run_id + 1) % 3
    change_next = jnp.concatenate([diff, jnp.ones((1,), bool)])
    pos = jnp.where(change_next, jnp.arange(1, nblk + 1), nblk + 1)
    ncp = lax.cummin(pos[::-1])[::-1]
    next_e = blk_expert[jnp.minimum(ncp, nblk - 1)]
    do_start = first_flag * (run_id != total_runs - 1).astype(jnp.int32)
    meta = jnp.stack([first_flag, slot, next_e, next_slot, blk_expert,
                      do_start]).astype(jnp.int32)

    grid_spec = pltpu.PrefetchScalarGridSpec(
        num_scalar_prefetch=1,
        grid=(nblk,),
        in_specs=[
            pl.BlockSpec((BR, DIM), lambda i, m: (i, 0)),
            pl.BlockSpec((BR, 128), lambda i, m: (i, 0)),
            pl.BlockSpec(memory_space=pltpu.HBM),
            pl.BlockSpec(memory_space=pltpu.HBM),
            pl.BlockSpec(memory_space=pltpu.HBM),
        ],
        out_specs=pl.BlockSpec((BR, DIM), lambda i, m: (i, 0)),
        scratch_shapes=[
            pltpu.VMEM((3, DIM, HIDDEN_DIM), jnp.float32),
            pltpu.VMEM((3, DIM, HIDDEN_DIM), jnp.float32),
            pltpu.VMEM((3, HIDDEN_DIM, DIM), jnp.float32),
            pltpu.SemaphoreType.DMA((3,)),
        ],
    )
    return pl.pallas_call(
        _expert_body,
        grid_spec=grid_spec,
        out_shape=jax.ShapeDtypeStruct((nblk * BR, DIM), jnp.float32),
    )(meta, perm, s16, w1, w3, w2)


def _combine(eo, dest_t, t):
    tok_per = t // NW
    tch = 8
    nch = tok_per // tch
    mesh = plsc.VectorSubcoreMesh(core_axis_name="c", subcore_axis_name="s",
                                  num_cores=NC, num_subcores=NS)

    @functools.partial(
        pl.kernel,
        out_type=jax.ShapeDtypeStruct((t, DIM), jnp.float32),
        mesh=mesh,
        scratch_types=[
            pltpu.VMEM((TOP_K, tok_per), jnp.int32),
            pltpu.VMEM((TOP_K * tch, DIM), jnp.float32),
            pltpu.VMEM((tch, DIM), jnp.float32),
            pltpu.SemaphoreType.DMA,
        ],
    )
    def combine(eo_hbm, dt_hbm, out_hbm, idx_v, rows_v, out_v, sem):
        wid = lax.axis_index("s") * NC + lax.axis_index("c")
        tbase = wid * tok_per
        for k in range(TOP_K):
            pltpu.sync_copy(dt_hbm.at[k, pl.ds(tbase, tok_per)], idx_v.at[k])

        def chunk_body(ci, carry):
            c0 = ci * tch
            cps = []
            for k in range(TOP_K):
                cps.append(pltpu.async_copy(
                    eo_hbm.at[idx_v.at[k, pl.ds(c0, tch)]],
                    rows_v.at[pl.ds(k * tch, tch)], sem))
            for cp in cps:
                cp.wait()

            def cbody(c, c2):
                sl = pl.ds(c * 16, 16)
                for tt in range(tch):
                    acc = rows_v[tt, sl]
                    for j in range(1, TOP_K):
                        acc = acc + rows_v[j * tch + tt, sl]
                    out_v[tt, sl] = acc
                return c2

            lax.fori_loop(0, DIM // 16, cbody, 0)
            pltpu.sync_copy(out_v, out_hbm.at[pl.ds(tbase + c0, tch)])
            return carry

        lax.fori_loop(0, nch, chunk_body, 0)

    return combine(eo, dest_t)


def kernel(x, gate_w, w1, w2, w3, expert_bias):
    bs, slen, dim = x.shape
    xf = x.reshape(-1, dim).astype(jnp.float32)
    t = xf.shape[0]
    tk = t * TOP_K
    nblk = tk // BR + NUM_EXPERTS
    cap = nblk * BR

    gwt = gate_w.T
    bias8 = jnp.broadcast_to(expert_bias[None, :], (8, NUM_EXPERTS))

    sel_t, w_t, rank_t, counts8 = _router(xf, gwt, bias8)

    counts = counts8[0]
    nblk_e = (counts + BR - 1) // BR
    cum_incl = jnp.cumsum(nblk_e)
    offset_pad = ((cum_incl - nblk_e) * BR).astype(jnp.int32)
    blk_expert = jnp.minimum(
        jnp.sum(cum_incl[:, None] <= jnp.arange(nblk)[None, :], axis=0),
        NUM_EXPERTS - 1).astype(jnp.int32)

    dest_t, w16 = _destcalc(sel_t, rank_t, w_t.reshape(-1, 1), offset_pad)

    perm, s16 = _dispatch(xf, dest_t, w16, cap)
    eo = _experts(blk_expert, perm, s16, w1, w2, w3, nblk)
    out = _combine(eo, dest_t, t)
    return out.reshape(bs, slen, dim)
